# baseline (speedup 1.0000x reference)
import functools

import jax
import jax.numpy as jnp
from jax import lax
from jax.experimental import pallas as pl
from jax.experimental.pallas import tpu as pltpu
from jax.experimental.pallas import tpu_sc as plsc


@functools.lru_cache
def _make_sc_gather(B, C):
    info = plsc.get_sparse_core_info()
    nc, ns, nl = info.num_cores, info.num_subcores, info.num_lanes
    nw = nc * ns
    assert B % nw == 0
    b_per_w = B // nw
    assert b_per_w % nl == 0

    @functools.partial(
        pl.kernel,
        mesh=plsc.VectorSubcoreMesh(core_axis_name="c", subcore_axis_name="s"),
        out_type=jax.ShapeDtypeStruct((B,), jnp.float32),
        scratch_types=[
            pltpu.VMEM((b_per_w,), jnp.int32),
            pltpu.VMEM((b_per_w,), jnp.float32),
            pltpu.SemaphoreType.DMA,
        ],
    )
    def gather_kernel(phi_hbm, labels_hbm, out_hbm, idx_v, vals_v, sem):
        wid = lax.axis_index("s") * nc + lax.axis_index("c")
        base = wid * b_per_w
        pltpu.sync_copy(labels_hbm.at[pl.ds(base, b_per_w)], idx_v)
        for j in range(b_per_w // nl):
            lbl = idx_v[pl.ds(j * nl, nl)]
            row = base + j * nl + lax.iota(jnp.int32, nl)
            idx_v[pl.ds(j * nl, nl)] = row * C + lbl
        pltpu.async_copy(phi_hbm.at[idx_v], vals_v, sem).wait()
        pltpu.sync_copy(vals_v, out_hbm.at[pl.ds(base, b_per_w)])

    return gather_kernel


@functools.lru_cache
def _make_tc_loss(B, C, bs):
    nblk = B // bs

    def body(cos_ref, p_ref, lbl_ref, out_ref):
        i = pl.program_id(0)
        cos = cos_ref[...]
        lbl = lbl_ref[...]
        p = p_ref[...]
        col = lax.broadcasted_iota(jnp.int32, (bs, C), 1)
        val = jnp.where(col == lbl, p, cos)
        m = jnp.max(val, axis=1, keepdims=True)
        s = jnp.sum(jnp.exp(val - m), axis=1, keepdims=True)
        part = jnp.sum(m + jnp.log(s) - p, keepdims=True)

        @pl.when(i == 0)
        def _init():
            out_ref[...] = jnp.zeros_like(out_ref)

        out_ref[...] += part

        @pl.when(i == nblk - 1)
        def _final():
            out_ref[...] = out_ref[...] / B

    return pl.pallas_call(
        body,
        grid=(nblk,),
        in_specs=[
            pl.BlockSpec((bs, C), lambda i: (i, 0)),
            pl.BlockSpec((bs, 1), lambda i: (i, 0)),
            pl.BlockSpec((bs, 1), lambda i: (i, 0)),
        ],
        out_specs=pl.BlockSpec((1, 1), lambda i: (0, 0)),
        out_shape=jax.ShapeDtypeStruct((1, 1), jnp.float32),
    )


def kernel(cos_theta, phi_theta, labels):
    B, C = cos_theta.shape
    p = _make_sc_gather(B, C)(phi_theta.reshape(-1), labels)
    out = _make_tc_loss(B, C, 512)(
        cos_theta, p.reshape(B, 1), labels.reshape(B, 1)
    )
    return out[0, 0]


# speedup vs baseline: 1.7112x; 1.7112x over previous
import functools

import jax
import jax.numpy as jnp
from jax import lax
from jax.experimental import pallas as pl


@functools.lru_cache
def _make_tc_loss(B, C, bs):
    nblk = B // bs

    def body(cos_ref, phi_ref, lbl_ref, out_ref):
        i = pl.program_id(0)
        cos = cos_ref[...]
        phi = phi_ref[...]
        lbl = lbl_ref[...]
        mask = lax.broadcasted_iota(jnp.int32, (bs, C), 1) == lbl
        val = jnp.where(mask, phi, cos)
        p = jnp.sum(jnp.where(mask, phi, 0.0), axis=1, keepdims=True)
        m = jnp.max(val, axis=1, keepdims=True)
        s = jnp.sum(jnp.exp(val - m), axis=1, keepdims=True)
        part = jnp.sum(m + jnp.log(s) - p, keepdims=True)

        @pl.when(i == 0)
        def _init():
            out_ref[...] = jnp.zeros_like(out_ref)

        out_ref[...] += part

        @pl.when(i == nblk - 1)
        def _final():
            out_ref[...] = out_ref[...] / B

    return pl.pallas_call(
        body,
        grid=(nblk,),
        in_specs=[
            pl.BlockSpec((bs, C), lambda i: (i, 0)),
            pl.BlockSpec((bs, C), lambda i: (i, 0)),
            pl.BlockSpec((bs, 1), lambda i: (i, 0)),
        ],
        out_specs=pl.BlockSpec((1, 1), lambda i: (0, 0)),
        out_shape=jax.ShapeDtypeStruct((1, 1), jnp.float32),
    )


def kernel(cos_theta, phi_theta, labels):
    B, C = cos_theta.shape
    out = _make_tc_loss(B, C, 512)(cos_theta, phi_theta, labels.reshape(B, 1))
    return out[0, 0]


# speedup vs baseline: 5.9188x; 3.4590x over previous
import functools

import jax
import jax.numpy as jnp
from jax import lax
from jax.experimental import pallas as pl


@functools.lru_cache
def _make_tc_loss(B, C, bs):
    nblk = B // bs

    def body(cos_ref, phi_ref, lbl_ref, out_ref):
        i = pl.program_id(0)
        cos = cos_ref[...]
        phi = phi_ref[...]
        lbl = lbl_ref[...]
        mask = lax.broadcasted_iota(jnp.int32, (C, bs), 0) == lbl
        val = jnp.where(mask, phi, cos)
        p = jnp.sum(jnp.where(mask, phi, 0.0), axis=0, keepdims=True)
        m = jnp.max(val, axis=0, keepdims=True)
        s = jnp.sum(jnp.exp(val - m), axis=0, keepdims=True)
        part = jnp.sum(m + jnp.log(s) - p, keepdims=True)

        @pl.when(i == 0)
        def _init():
            out_ref[...] = jnp.zeros_like(out_ref)

        out_ref[...] += part

        @pl.when(i == nblk - 1)
        def _final():
            out_ref[...] = out_ref[...] / B

    return pl.pallas_call(
        body,
        grid=(nblk,),
        in_specs=[
            pl.BlockSpec((C, bs), lambda i: (0, i)),
            pl.BlockSpec((C, bs), lambda i: (0, i)),
            pl.BlockSpec((1, bs), lambda i: (0, i)),
        ],
        out_specs=pl.BlockSpec((1, 1), lambda i: (0, 0)),
        out_shape=jax.ShapeDtypeStruct((1, 1), jnp.float32),
    )


def kernel(cos_theta, phi_theta, labels):
    B, C = cos_theta.shape
    out = _make_tc_loss(B, C, 512)(
        cos_theta.T, phi_theta.T, labels.reshape(1, B)
    )
    return out[0, 0]


# speedup vs baseline: 6.4656x; 1.0924x over previous
import functools

import jax
import jax.numpy as jnp
from jax import lax
from jax.experimental import pallas as pl


@functools.lru_cache
def _make_tc_loss(B, C, bs):
    nblk = B // bs

    def body(cos_ref, phi_ref, lbl_ref, out_ref):
        i = pl.program_id(0)
        cos = cos_ref[...]
        phi = phi_ref[...]
        lbl = lbl_ref[...]
        mask = lax.broadcasted_iota(jnp.int32, (C, bs), 0) == lbl
        val = jnp.where(mask, phi, cos)
        p = jnp.sum(jnp.where(mask, phi, 0.0), axis=0, keepdims=True)
        m = jnp.max(val, axis=0, keepdims=True)
        s = jnp.sum(jnp.exp(val - m), axis=0, keepdims=True)
        part = jnp.sum(m + jnp.log(s) - p, keepdims=True)

        @pl.when(i == 0)
        def _init():
            out_ref[...] = jnp.zeros_like(out_ref)

        out_ref[...] += part

        @pl.when(i == nblk - 1)
        def _final():
            out_ref[...] = out_ref[...] / B

    return pl.pallas_call(
        body,
        grid=(nblk,),
        in_specs=[
            pl.BlockSpec((C, bs), lambda i: (0, i)),
            pl.BlockSpec((C, bs), lambda i: (0, i)),
            pl.BlockSpec((1, bs), lambda i: (0, i)),
        ],
        out_specs=pl.BlockSpec((1, 1), lambda i: (0, 0)),
        out_shape=jax.ShapeDtypeStruct((1, 1), jnp.float32),
    )


def kernel(cos_theta, phi_theta, labels):
    B, C = cos_theta.shape
    out = _make_tc_loss(B, C, 1024)(
        cos_theta.T, phi_theta.T, labels.reshape(1, B)
    )
    return out[0, 0]
